# baseline (speedup 1.0000x reference)
import functools

import jax
import jax.numpy as jnp
from jax import lax
from jax.experimental import pallas as pl
from jax.experimental.pallas import tpu as pltpu
from jax.experimental.pallas import tpu_sc as plsc

_SMOOTHING = 0.1
_CONFIDENCE = 1.0 - _SMOOTHING
_V = 100000
_B = 1024
_FILL = _SMOOTHING / (_V - 2)

_TOTAL = _B * _V
_NCHUNK = 25
_CHUNK = _TOTAL // _NCHUNK
_INIT_SLICE = 8192
_L = 16
_NC, _NS = 2, 16
_NW = _NC * _NS
_PER_W = _B // _NW


def _fill_body(out_ref, scratch, sem):
    def init(i, carry):
        scratch[pl.ds(i * _INIT_SLICE, _INIT_SLICE)] = jnp.full(
            (_INIT_SLICE,), _FILL, jnp.float32)
        return carry

    lax.fori_loop(0, _CHUNK // _INIT_SLICE, init, 0)
    for j in range(_NCHUNK):
        pltpu.make_async_copy(
            scratch, out_ref.at[pl.ds(j * _CHUNK, _CHUNK)], sem).start()
    for j in range(_NCHUNK):
        pltpu.make_async_copy(
            scratch, out_ref.at[pl.ds(0, _CHUNK)], sem).wait()


_fill = pl.pallas_call(
    _fill_body,
    out_shape=jax.ShapeDtypeStruct((_TOTAL,), jnp.float32),
    out_specs=pl.BlockSpec(memory_space=pl.ANY),
    scratch_shapes=[
        pltpu.VMEM((_CHUNK,), jnp.float32),
        pltpu.SemaphoreType.DMA,
    ],
)


@functools.partial(
    pl.kernel,
    mesh=plsc.VectorSubcoreMesh(core_axis_name="c", subcore_axis_name="s"),
    scratch_types=[
        pltpu.VMEM((_PER_W,), jnp.int32),
        pltpu.VMEM((2 * _PER_W,), jnp.int32),
        pltpu.VMEM((2 * _PER_W,), jnp.float32),
        pltpu.SemaphoreType.DMA,
    ],
)
def _sc_scatter(ids_hbm, out_hbm, ids_v, idx_v, val_v, sem):
    wid = lax.axis_index("s") * _NC + lax.axis_index("c")
    base = wid * _PER_W
    pltpu.async_copy(ids_hbm.at[pl.ds(base, _PER_W)], ids_v, sem).wait()
    zero = jnp.zeros((_L,), jnp.float32)
    conf = jnp.full((_L,), _CONFIDENCE, jnp.float32)
    for g in range(_PER_W // _L):
        ids = ids_v[pl.ds(g * _L, _L)]
        rows = lax.iota(jnp.int32, _L) + (base + g * _L)
        idx_v[pl.ds(g * _L, _L)] = rows * _V + ids
        idx_v[pl.ds(_PER_W + g * _L, _L)] = rows * _V
        val_v[pl.ds(g * _L, _L)] = jnp.where(ids == 0, zero, conf)
        val_v[pl.ds(_PER_W + g * _L, _L)] = zero
    pltpu.async_copy(val_v, out_hbm.at[idx_v], sem).wait()


@jax.jit
def _run(ids):
    flat = _fill()
    buf = jax.new_ref(flat)
    _sc_scatter(ids, buf)
    return jax.freeze(buf).reshape(_B, _V)


def kernel(tgt_tok_id):
    return _run(tgt_tok_id.reshape(-1).astype(jnp.int32))
